# baseline (speedup 1.0000x reference)
import functools

import jax
import jax.numpy as jnp
from jax import lax
from jax.experimental import pallas as pl
from jax.experimental.pallas import tpu as pltpu
from jax.experimental.pallas import tpu_sc as plsc

_N_RAYS = 16384
_S = 64
_LAMBDA_OPACITY = 0.001
_LAMBDA_DISTORTION = 0.001

_NC = 2
_NS = 16
_L = 16
_NW = _NC * _NS
_RAYS_PER_W = _N_RAYS // _NW
_CHUNK_RAYS = 32
_CHUNK = _CHUNK_RAYS * _S
_NCHUNK = _RAYS_PER_W // _CHUNK_RAYS
_GROUPS = _CHUNK_RAYS // _L


def _dist_body(ws_hbm, ts_hbm, ds_hbm, out_hbm,
               w0, t0, d0, w1, t1, d1, outv, sem0, sem1):
    wid = lax.axis_index("s") * _NC + lax.axis_index("c")
    base = wid * _RAYS_PER_W * _S
    bufs = ((w0, t0, d0, sem0), (w1, t1, d1, sem1))

    def issue(c):
        wb, tb, db, sem = bufs[c % 2]
        off = base + c * _CHUNK
        return (pltpu.async_copy(ws_hbm.at[pl.ds(off, _CHUNK)], wb, sem),
                pltpu.async_copy(ts_hbm.at[pl.ds(off, _CHUNK)], tb, sem),
                pltpu.async_copy(ds_hbm.at[pl.ds(off, _CHUNK)], db, sem))

    pending = issue(0)
    for c in range(_NCHUNK):
        nxt = issue(c + 1) if c + 1 < _NCHUNK else None
        for h in pending:
            h.wait()
        pending = nxt
        wb, tb, db, _ = bufs[c % 2]

        @plsc.parallel_loop(0, _CHUNK_RAYS, unroll=2)
        def _ray(r, _wb=wb, _tb=tb, _db=db, _c=c):
            off = r * _S
            acc_a = jnp.zeros((_L,), jnp.float32)
            acc_b = jnp.zeros((_L,), jnp.float32)
            cw = jnp.float32(0.0)
            cwt = jnp.float32(0.0)
            for k in range(_S // _L):
                sl = pl.ds(off + k * _L, _L)
                w = _wb[sl]
                t = _tb[sl]
                dl = _db[sl]
                wt = w * t
                iw = plsc.cumsum(w)
                iwt = plsc.cumsum(wt)
                w_ex = (iw - w) + cw
                wt_ex = (iwt - wt) + cwt
                acc_a = acc_a + w * (t * w_ex - wt_ex)
                acc_b = acc_b + (w * w) * dl
                if k + 1 < _S // _L:
                    cw = cw + iw[_L - 1]
                    cwt = cwt + iwt[_L - 1]
            tot = jnp.sum(2.0 * acc_a + acc_b * (1.0 / 3.0))
            tot = tot * _LAMBDA_DISTORTION
            lane = lax.iota(jnp.int32, _L)
            plsc.store_scatter(
                outv,
                [jnp.full((_L,), _c * _CHUNK_RAYS + r, jnp.int32)],
                jnp.full((_L,), tot, jnp.float32),
                mask=lane == 0)

    pltpu.sync_copy(outv, out_hbm.at[pl.ds(wid * _RAYS_PER_W, _RAYS_PER_W)])


_dist_call = functools.partial(
    pl.kernel,
    out_type=jax.ShapeDtypeStruct((_N_RAYS,), jnp.float32),
    mesh=plsc.VectorSubcoreMesh(core_axis_name="c", subcore_axis_name="s"),
    compiler_params=pltpu.CompilerParams(needs_layout_passes=False,
                                         use_tc_tiling_on_sc=False),
    scratch_types=[
        pltpu.VMEM((_CHUNK,), jnp.float32),
        pltpu.VMEM((_CHUNK,), jnp.float32),
        pltpu.VMEM((_CHUNK,), jnp.float32),
        pltpu.VMEM((_CHUNK,), jnp.float32),
        pltpu.VMEM((_CHUNK,), jnp.float32),
        pltpu.VMEM((_CHUNK,), jnp.float32),
        pltpu.VMEM((_RAYS_PER_W,), jnp.float32),
        pltpu.SemaphoreType.DMA,
        pltpu.SemaphoreType.DMA,
    ],
)(_dist_body)


def _loss_body(rep_ref, rgb_ref, tgt_ref, opac_ref, lam_ref,
               loss_ref, dopa_ref):
    rep = rep_ref[...]
    mn = (rep == 0).astype(jnp.float32)
    mo = (rep == 1).astype(jnp.float32)
    n_new = jnp.sum(mn)
    n_old = jnp.sum(mo)
    diff = rgb_ref[...] - tgt_ref[...]
    sq = diff * diff
    se = jnp.sum(sq, axis=0, keepdims=True)
    charb = jnp.sum(jnp.sqrt(sq + 1e-6), axis=0, keepdims=True)
    loss = jnp.sum(se * mn) / n_new
    old_term = jnp.sum(charb * mo) * lam_ref[0] / jnp.maximum(n_old, 1.0)
    loss_ref[0, 0] = loss + jnp.where(n_old > 0, old_term, 0.0)
    o = opac_ref[...] + 1e-10
    dopa_ref[...] = _LAMBDA_OPACITY * (-o * jnp.log(o))


_loss_call = pl.pallas_call(
    _loss_body,
    out_shape=(
        jax.ShapeDtypeStruct((1, 1), jnp.float32),
        jax.ShapeDtypeStruct((1, _N_RAYS), jnp.float32),
    ),
    in_specs=[
        pl.BlockSpec(memory_space=pltpu.VMEM),
        pl.BlockSpec(memory_space=pltpu.VMEM),
        pl.BlockSpec(memory_space=pltpu.VMEM),
        pl.BlockSpec(memory_space=pltpu.VMEM),
        pl.BlockSpec(memory_space=pltpu.SMEM),
    ],
    out_specs=(
        pl.BlockSpec(memory_space=pltpu.SMEM),
        pl.BlockSpec(memory_space=pltpu.VMEM),
    ),
)


def kernel(results_rgb, results_opacity, results_ws, results_deltas,
           results_ts, rays_a, target_rgb, target_is_rep, lambda_p):
    d_distortion = _dist_call(results_ws, results_ts, results_deltas)
    lam = jnp.asarray(lambda_p, jnp.float32).reshape(1)
    loss2, dopa2 = _loss_call(
        target_is_rep.reshape(1, _N_RAYS),
        results_rgb.T,
        target_rgb.T,
        results_opacity.reshape(1, _N_RAYS),
        lam,
    )
    return (loss2.reshape(()), dopa2.reshape(_N_RAYS), d_distortion)


# speedup vs baseline: 1.2075x; 1.2075x over previous
import functools

import jax
import jax.numpy as jnp
from jax import lax
from jax.experimental import pallas as pl
from jax.experimental.pallas import tpu as pltpu
from jax.experimental.pallas import tpu_sc as plsc

_N_RAYS = 16384
_S = 64
_LAMBDA_OPACITY = 0.001
_LAMBDA_DISTORTION = 0.001

_NC = 2
_NS = 16
_L = 16
_NW = _NC * _NS
_RAYS_PER_W = _N_RAYS // _NW
_CHUNK_RAYS = 64
_CHUNK = _CHUNK_RAYS * _S
_NCHUNK = _RAYS_PER_W // _CHUNK_RAYS
_GROUPS = _CHUNK_RAYS // _L


def _dist_body(ws_hbm, ts_hbm, ds_hbm, out_hbm,
               w0, t0, d0, w1, t1, d1, outv, sem0, sem1):
    wid = lax.axis_index("s") * _NC + lax.axis_index("c")
    base = wid * _RAYS_PER_W * _S
    bufs = ((w0, t0, d0, sem0), (w1, t1, d1, sem1))

    def issue(c):
        wb, tb, db, sem = bufs[c % 2]
        off = base + c * _CHUNK
        return (pltpu.async_copy(ws_hbm.at[pl.ds(off, _CHUNK)], wb, sem),
                pltpu.async_copy(ts_hbm.at[pl.ds(off, _CHUNK)], tb, sem),
                pltpu.async_copy(ds_hbm.at[pl.ds(off, _CHUNK)], db, sem))

    pending = issue(0)
    for c in range(_NCHUNK):
        nxt = issue(c + 1) if c + 1 < _NCHUNK else None
        for h in pending:
            h.wait()
        pending = nxt
        wb, tb, db, _ = bufs[c % 2]

        @plsc.parallel_loop(0, _CHUNK_RAYS, unroll=2)
        def _ray(r, _wb=wb, _tb=tb, _db=db, _c=c):
            off = r * _S
            acc_a = jnp.zeros((_L,), jnp.float32)
            acc_b = jnp.zeros((_L,), jnp.float32)
            cw = jnp.float32(0.0)
            cwt = jnp.float32(0.0)
            corr = jnp.float32(0.0)
            for k in range(_S // _L):
                sl = pl.ds(off + k * _L, _L)
                w = _wb[sl]
                t = _tb[sl]
                dl = _db[sl]
                wt = w * t
                iw = plsc.cumsum(w)
                iwt = plsc.cumsum(wt)
                acc_a = acc_a + w * (t * (iw - w) - (iwt - wt))
                acc_b = acc_b + (w * w) * dl
                totw = iw[_L - 1]
                totwt = iwt[_L - 1]
                corr = corr + (cw * totwt - cwt * totw)
                if k + 1 < _S // _L:
                    cw = cw + totw
                    cwt = cwt + totwt
            tot = jnp.sum(2.0 * acc_a + acc_b * (1.0 / 3.0)) + 2.0 * corr
            tot = tot * _LAMBDA_DISTORTION
            lane = lax.iota(jnp.int32, _L)
            plsc.store_scatter(
                outv,
                [jnp.full((_L,), _c * _CHUNK_RAYS + r, jnp.int32)],
                jnp.full((_L,), tot, jnp.float32),
                mask=lane == 0)

    pltpu.sync_copy(outv, out_hbm.at[pl.ds(wid * _RAYS_PER_W, _RAYS_PER_W)])


_dist_call = functools.partial(
    pl.kernel,
    out_type=jax.ShapeDtypeStruct((_N_RAYS,), jnp.float32),
    mesh=plsc.VectorSubcoreMesh(core_axis_name="c", subcore_axis_name="s"),
    compiler_params=pltpu.CompilerParams(needs_layout_passes=False,
                                         use_tc_tiling_on_sc=False),
    scratch_types=[
        pltpu.VMEM((_CHUNK,), jnp.float32),
        pltpu.VMEM((_CHUNK,), jnp.float32),
        pltpu.VMEM((_CHUNK,), jnp.float32),
        pltpu.VMEM((_CHUNK,), jnp.float32),
        pltpu.VMEM((_CHUNK,), jnp.float32),
        pltpu.VMEM((_CHUNK,), jnp.float32),
        pltpu.VMEM((_RAYS_PER_W,), jnp.float32),
        pltpu.SemaphoreType.DMA,
        pltpu.SemaphoreType.DMA,
    ],
)(_dist_body)


def _loss_body(rep_ref, rgb_ref, tgt_ref, opac_ref, lam_ref,
               loss_ref, dopa_ref):
    rep = rep_ref[...]
    mn = (rep == 0).astype(jnp.float32)
    mo = (rep == 1).astype(jnp.float32)
    n_new = jnp.sum(mn)
    n_old = jnp.sum(mo)
    diff = rgb_ref[...] - tgt_ref[...]
    sq = diff * diff
    se = jnp.sum(sq, axis=0, keepdims=True)
    charb = jnp.sum(jnp.sqrt(sq + 1e-6), axis=0, keepdims=True)
    loss = jnp.sum(se * mn) / n_new
    old_term = jnp.sum(charb * mo) * lam_ref[0] / jnp.maximum(n_old, 1.0)
    loss_ref[0, 0] = loss + jnp.where(n_old > 0, old_term, 0.0)
    o = opac_ref[...] + 1e-10
    dopa_ref[...] = _LAMBDA_OPACITY * (-o * jnp.log(o))


_loss_call = pl.pallas_call(
    _loss_body,
    out_shape=(
        jax.ShapeDtypeStruct((1, 1), jnp.float32),
        jax.ShapeDtypeStruct((1, _N_RAYS), jnp.float32),
    ),
    in_specs=[
        pl.BlockSpec(memory_space=pltpu.VMEM),
        pl.BlockSpec(memory_space=pltpu.VMEM),
        pl.BlockSpec(memory_space=pltpu.VMEM),
        pl.BlockSpec(memory_space=pltpu.VMEM),
        pl.BlockSpec(memory_space=pltpu.SMEM),
    ],
    out_specs=(
        pl.BlockSpec(memory_space=pltpu.SMEM),
        pl.BlockSpec(memory_space=pltpu.VMEM),
    ),
)


def kernel(results_rgb, results_opacity, results_ws, results_deltas,
           results_ts, rays_a, target_rgb, target_is_rep, lambda_p):
    d_distortion = _dist_call(results_ws, results_ts, results_deltas)
    lam = jnp.asarray(lambda_p, jnp.float32).reshape(1)
    loss2, dopa2 = _loss_call(
        target_is_rep.reshape(1, _N_RAYS),
        results_rgb.T,
        target_rgb.T,
        results_opacity.reshape(1, _N_RAYS),
        lam,
    )
    return (loss2.reshape(()), dopa2.reshape(_N_RAYS), d_distortion)
